# initial kernel scaffold (unmeasured)
import jax
import jax.numpy as jnp
from jax import lax
from jax.experimental import pallas as pl
from jax.experimental.pallas import tpu as pltpu

N_DEV = 4


def kernel(x, router_W, route_idx, expert_W):
    n_tok, d_in = x.shape
    e_loc, _, d_out = expert_W.shape
    chunk = n_tok // N_DEV

    def body(x_ref, rw_ref, idx_ref, ew_ref, out_ref,
             xbf_ref, g0_ref, g1_ref, acc_ref,
             send_buf, recv_buf, send_sems, recv_sems):
        e = pl.program_id(0)
        my = lax.axis_index("i")
        right = lax.rem(my + 1, N_DEV)
        left = lax.rem(my + N_DEV - 1, N_DEV)

        @pl.when(e == 0)
        def _init():
            xf = x_ref[...]
            xbf_ref[...] = xf.astype(jnp.bfloat16)
            scores = jnp.dot(xf, rw_ref[...],
                             preferred_element_type=jnp.float32)
            m = jnp.max(scores, axis=-1, keepdims=True)
            p = jnp.exp(scores - m)
            iota = lax.broadcasted_iota(jnp.int32, scores.shape, 1)
            p0 = jnp.sum(jnp.where(iota == idx_ref[:, 0:1], p, 0.0),
                         axis=1, keepdims=True)
            p1 = jnp.sum(jnp.where(iota == idx_ref[:, 1:2], p, 0.0),
                         axis=1, keepdims=True)
            s = p0 + p1
            g0_ref[...] = p0 / s
            g1_ref[...] = p1 / s
            acc_ref[...] = jnp.zeros_like(acc_ref)

        eid = my * e_loc + e
        w_col = (jnp.where(idx_ref[:, 0:1] == eid, g0_ref[...], 0.0)
                 + jnp.where(idx_ref[:, 1:2] == eid, g1_ref[...], 0.0))
        wblk = ew_ref[0, :, :].astype(jnp.bfloat16)
        y = jnp.dot(xbf_ref[...], wblk, preferred_element_type=jnp.float32)
        acc_ref[...] += w_col * y

        @pl.when(e == pl.num_programs(0) - 1)
        def _allreduce():
            barrier_sem = pltpu.get_barrier_semaphore()
            for nbr in (left, right):
                pl.semaphore_signal(barrier_sem, inc=1, device_id=(nbr,),
                                    device_id_type=pl.DeviceIdType.MESH)
            pl.semaphore_wait(barrier_sem, 2)

            def acc_chunk(c):
                return acc_ref[pl.ds(c * chunk, chunk), :]

            def hop(h, src):
                rdma = pltpu.make_async_remote_copy(
                    src_ref=src,
                    dst_ref=recv_buf.at[h],
                    send_sem=send_sems.at[h],
                    recv_sem=recv_sems.at[h],
                    device_id=(right,),
                    device_id_type=pl.DeviceIdType.MESH,
                )
                rdma.start()
                rdma.wait()

            send_buf[0, :, :] = acc_chunk(my).astype(jnp.bfloat16)
            for h in range(N_DEV - 1):
                hop(h, send_buf.at[h])
                r = lax.rem(my + (N_DEV - h - 1), N_DEV)
                ssum = acc_chunk(r) + recv_buf[h, :, :].astype(jnp.float32)
                if h < N_DEV - 2:
                    send_buf[h + 1, :, :] = ssum.astype(jnp.bfloat16)
                else:
                    out_ref[pl.ds(r * chunk, chunk), :] = ssum
                    send_buf[N_DEV - 1, :, :] = ssum.astype(jnp.bfloat16)

            for t in range(N_DEV - 1):
                h = (N_DEV - 1) + t
                src = send_buf.at[N_DEV - 1] if t == 0 else recv_buf.at[h - 1]
                hop(h, src)
                c = lax.rem(my + (N_DEV - t), N_DEV)
                out_ref[pl.ds(c * chunk, chunk), :] = \
                    recv_buf[h, :, :].astype(jnp.float32)

    return pl.pallas_call(
        body,
        grid=(e_loc,),
        out_shape=jax.ShapeDtypeStruct((n_tok, d_out), jnp.float32),
        in_specs=[
            pl.BlockSpec((n_tok, d_in), lambda e: (0, 0)),
            pl.BlockSpec(router_W.shape, lambda e: (0, 0)),
            pl.BlockSpec(route_idx.shape, lambda e: (0, 0)),
            pl.BlockSpec((1,) + expert_W.shape[1:], lambda e: (e, 0, 0)),
        ],
        out_specs=pl.BlockSpec((n_tok, d_out), lambda e: (0, 0)),
        scratch_shapes=[
            pltpu.VMEM((n_tok, d_in), jnp.bfloat16),
            pltpu.VMEM((n_tok, 1), jnp.float32),
            pltpu.VMEM((n_tok, 1), jnp.float32),
            pltpu.VMEM((n_tok, d_out), jnp.float32),
            pltpu.VMEM((N_DEV, chunk, d_out), jnp.bfloat16),
            pltpu.VMEM((2 * (N_DEV - 1), chunk, d_out), jnp.bfloat16),
            pltpu.SemaphoreType.DMA((2 * (N_DEV - 1),)),
            pltpu.SemaphoreType.DMA((2 * (N_DEV - 1),)),
        ],
        compiler_params=pltpu.CompilerParams(
            collective_id=0,
            dimension_semantics=("arbitrary",),
        ),
    )(x, router_W, route_idx, expert_W)


# baseline (device time: 143178 ns/iter reference)
import jax
import jax.numpy as jnp
from jax import lax
from jax.experimental import pallas as pl
from jax.experimental.pallas import tpu as pltpu

N_DEV = 4


def kernel(x, router_W, route_idx, expert_W):
    n_tok, d_in = x.shape
    e_loc, _, d_out = expert_W.shape
    chunk = n_tok // N_DEV

    def body(x_ref, rw_ref, idx_ref, ew_ref, out_ref,
             xbf_ref, g0_ref, g1_ref, acc_ref,
             send_buf, recv_buf, send_sems, recv_sems):
        e = pl.program_id(0)
        my = lax.axis_index("i")
        right = lax.rem(my + 1, N_DEV)
        left = lax.rem(my + N_DEV - 1, N_DEV)

        @pl.when(e == 0)
        def _init():
            xf = x_ref[...]
            xbf_ref[...] = xf.astype(jnp.bfloat16)
            scores = jnp.dot(xf, rw_ref[...],
                             preferred_element_type=jnp.float32)
            m = jnp.max(scores, axis=-1, keepdims=True)
            p = jnp.exp(scores - m)
            iota = lax.broadcasted_iota(jnp.int32, scores.shape, 1)
            p0 = jnp.sum(jnp.where(iota == idx_ref[:, 0:1], p, 0.0),
                         axis=1, keepdims=True)
            p1 = jnp.sum(jnp.where(iota == idx_ref[:, 1:2], p, 0.0),
                         axis=1, keepdims=True)
            s = p0 + p1
            g0_ref[...] = p0 / s
            g1_ref[...] = p1 / s
            acc_ref[...] = jnp.zeros_like(acc_ref)

        eid = my * e_loc + e
        w_col = (jnp.where(idx_ref[:, 0:1] == eid, g0_ref[...], 0.0)
                 + jnp.where(idx_ref[:, 1:2] == eid, g1_ref[...], 0.0))
        wblk = ew_ref[0, :, :].astype(jnp.bfloat16)
        y = jnp.dot(xbf_ref[...], wblk, preferred_element_type=jnp.float32)
        acc_ref[...] += w_col * y

        @pl.when(e == pl.num_programs(0) - 1)
        def _allreduce():
            barrier_sem = pltpu.get_barrier_semaphore()
            for nbr in (left, right):
                pl.semaphore_signal(barrier_sem, inc=1, device_id=(nbr,),
                                    device_id_type=pl.DeviceIdType.MESH)
            pl.semaphore_wait(barrier_sem, 2)

            def acc_chunk(c):
                return acc_ref[pl.ds(c * chunk, chunk), :]

            def hop(h, src):
                rdma = pltpu.make_async_remote_copy(
                    src_ref=src,
                    dst_ref=recv_buf.at[h],
                    send_sem=send_sems.at[h],
                    recv_sem=recv_sems.at[h],
                    device_id=(right,),
                    device_id_type=pl.DeviceIdType.MESH,
                )
                rdma.start()
                rdma.wait()

            send_buf[0, :, :] = acc_chunk(my).astype(jnp.bfloat16)
            for h in range(N_DEV - 1):
                hop(h, send_buf.at[h])
                r = lax.rem(my + (N_DEV - h - 1), N_DEV)
                ssum = acc_chunk(r) + recv_buf[h, :, :].astype(jnp.float32)
                if h < N_DEV - 2:
                    send_buf[h + 1, :, :] = ssum.astype(jnp.bfloat16)
                else:
                    out_ref[pl.ds(r * chunk, chunk), :] = ssum
                    send_buf[N_DEV - 1, :, :] = ssum.astype(jnp.bfloat16)

            for t in range(N_DEV - 1):
                h = (N_DEV - 1) + t
                src = send_buf.at[N_DEV - 1] if t == 0 else recv_buf.at[h - 1]
                hop(h, src)
                c = lax.rem(my + (N_DEV - t), N_DEV)
                out_ref[pl.ds(c * chunk, chunk), :] = \
                    recv_buf[h, :, :].astype(jnp.float32)

    return pl.pallas_call(
        body,
        grid=(e_loc,),
        out_shape=jax.ShapeDtypeStruct((n_tok, d_out), jnp.float32),
        in_specs=[
            pl.BlockSpec((n_tok, d_in), lambda e: (0, 0)),
            pl.BlockSpec(router_W.shape, lambda e: (0, 0)),
            pl.BlockSpec(route_idx.shape, lambda e: (0, 0)),
            pl.BlockSpec((1,) + expert_W.shape[1:], lambda e: (e, 0, 0)),
        ],
        out_specs=pl.BlockSpec((n_tok, d_out), lambda e: (0, 0)),
        scratch_shapes=[
            pltpu.VMEM((n_tok, d_in), jnp.bfloat16),
            pltpu.VMEM((n_tok, 1), jnp.float32),
            pltpu.VMEM((n_tok, 1), jnp.float32),
            pltpu.VMEM((n_tok, d_out), jnp.float32),
            pltpu.VMEM((N_DEV, chunk, d_out), jnp.bfloat16),
            pltpu.VMEM((2 * (N_DEV - 1), chunk, d_out), jnp.bfloat16),
            pltpu.SemaphoreType.DMA((2 * (N_DEV - 1),)),
            pltpu.SemaphoreType.DMA((2 * (N_DEV - 1),)),
        ],
        compiler_params=pltpu.CompilerParams(
            collective_id=0,
            dimension_semantics=("arbitrary",),
            vmem_limit_bytes=100 * 1024 * 1024,
        ),
    )(x, router_W, route_idx, expert_W)


# device time: 114752 ns/iter; 1.2477x vs baseline; 1.2477x over previous
import jax
import jax.numpy as jnp
from jax import lax
from jax.experimental import pallas as pl
from jax.experimental.pallas import tpu as pltpu

N_DEV = 4


def kernel(x, router_W, route_idx, expert_W):
    n_tok, d_in = x.shape
    e_loc, _, d_out = expert_W.shape
    chunk = n_tok // N_DEV

    def body(x_ref, rw_ref, idx_ref, ew_ref, out_ref,
             xbf_ref, g0_ref, g1_ref, acc_ref,
             send_buf, recv_buf, send_sems, recv_sems):
        e = pl.program_id(0)
        my = lax.axis_index("i")
        right = lax.rem(my + 1, N_DEV)
        left = lax.rem(my + N_DEV - 1, N_DEV)

        @pl.when(e == 0)
        def _init():
            xf = x_ref[...]
            xbf_ref[...] = xf.astype(jnp.bfloat16)
            scores = jnp.dot(xf, rw_ref[...],
                             preferred_element_type=jnp.float32)
            m = jnp.max(scores, axis=-1, keepdims=True)
            p = jnp.exp(scores - m)
            iota = lax.broadcasted_iota(jnp.int32, scores.shape, 1)
            p0 = jnp.sum(jnp.where(iota == idx_ref[:, 0:1], p, 0.0),
                         axis=1, keepdims=True)
            p1 = jnp.sum(jnp.where(iota == idx_ref[:, 1:2], p, 0.0),
                         axis=1, keepdims=True)
            s = p0 + p1
            g0_ref[...] = p0 / s
            g1_ref[...] = p1 / s
            acc_ref[...] = jnp.zeros_like(acc_ref)

        eid = my * e_loc + e
        w_col = (jnp.where(idx_ref[:, 0:1] == eid, g0_ref[...], 0.0)
                 + jnp.where(idx_ref[:, 1:2] == eid, g1_ref[...], 0.0))
        wblk = ew_ref[0, :, :].astype(jnp.bfloat16)
        y = jnp.dot(xbf_ref[...], wblk, preferred_element_type=jnp.float32)
        acc_ref[...] += w_col * y

        @pl.when(e == pl.num_programs(0) - 1)
        def _allreduce():
            barrier_sem = pltpu.get_barrier_semaphore()
            for k in range(1, N_DEV):
                pl.semaphore_signal(
                    barrier_sem, inc=1,
                    device_id=(lax.rem(my + k, N_DEV),),
                    device_id_type=pl.DeviceIdType.MESH)
            pl.semaphore_wait(barrier_sem, N_DEV - 1)

            def acc_chunk(c):
                return acc_ref[pl.ds(c * chunk, chunk), :]

            rs = []
            for k in range(1, N_DEV):
                tgt = lax.rem(my + k, N_DEV)
                send_buf[k - 1, :, :] = acc_chunk(tgt).astype(jnp.bfloat16)
                rdma = pltpu.make_async_remote_copy(
                    src_ref=send_buf.at[k - 1],
                    dst_ref=recv_buf.at[k - 1],
                    send_sem=send_sems.at[k - 1],
                    recv_sem=recv_sems.at[k - 1],
                    device_id=(tgt,),
                    device_id_type=pl.DeviceIdType.MESH,
                )
                rdma.start()
                rs.append(rdma)
            for rdma in rs:
                rdma.wait()
            ssum = acc_chunk(my)
            for k in range(1, N_DEV):
                ssum = ssum + recv_buf[k - 1, :, :].astype(jnp.float32)
            out_ref[pl.ds(my * chunk, chunk), :] = ssum
            send_buf[N_DEV - 1, :, :] = ssum.astype(jnp.bfloat16)

            ag = []
            for k in range(1, N_DEV):
                tgt = lax.rem(my + k, N_DEV)
                rdma = pltpu.make_async_remote_copy(
                    src_ref=send_buf.at[N_DEV - 1],
                    dst_ref=recv_buf.at[N_DEV - 2 + k],
                    send_sem=send_sems.at[N_DEV - 2 + k],
                    recv_sem=recv_sems.at[N_DEV - 2 + k],
                    device_id=(tgt,),
                    device_id_type=pl.DeviceIdType.MESH,
                )
                rdma.start()
                ag.append(rdma)
            for k, rdma in zip(range(1, N_DEV), ag):
                rdma.wait()
                src_dev = lax.rem(my + (N_DEV - k), N_DEV)
                out_ref[pl.ds(src_dev * chunk, chunk), :] = \
                    recv_buf[N_DEV - 2 + k, :, :].astype(jnp.float32)

    return pl.pallas_call(
        body,
        grid=(e_loc,),
        out_shape=jax.ShapeDtypeStruct((n_tok, d_out), jnp.float32),
        in_specs=[
            pl.BlockSpec((n_tok, d_in), lambda e: (0, 0)),
            pl.BlockSpec(router_W.shape, lambda e: (0, 0)),
            pl.BlockSpec(route_idx.shape, lambda e: (0, 0)),
            pl.BlockSpec((1,) + expert_W.shape[1:], lambda e: (e, 0, 0)),
        ],
        out_specs=pl.BlockSpec((n_tok, d_out), lambda e: (0, 0)),
        scratch_shapes=[
            pltpu.VMEM((n_tok, d_in), jnp.bfloat16),
            pltpu.VMEM((n_tok, 1), jnp.float32),
            pltpu.VMEM((n_tok, 1), jnp.float32),
            pltpu.VMEM((n_tok, d_out), jnp.float32),
            pltpu.VMEM((N_DEV, chunk, d_out), jnp.bfloat16),
            pltpu.VMEM((2 * (N_DEV - 1), chunk, d_out), jnp.bfloat16),
            pltpu.SemaphoreType.DMA((2 * (N_DEV - 1),)),
            pltpu.SemaphoreType.DMA((2 * (N_DEV - 1),)),
        ],
        compiler_params=pltpu.CompilerParams(
            collective_id=0,
            dimension_semantics=("arbitrary",),
            vmem_limit_bytes=100 * 1024 * 1024,
        ),
    )(x, router_W, route_idx, expert_W)


# device time: 98173 ns/iter; 1.4584x vs baseline; 1.1689x over previous
import jax
import jax.numpy as jnp
from jax import lax
from jax.experimental import pallas as pl
from jax.experimental.pallas import tpu as pltpu

N_DEV = 4
N_Q = 4


def kernel(x, router_W, route_idx, expert_W):
    n_tok, d_in = x.shape
    e_loc, _, d_out = expert_W.shape
    qrows = n_tok // N_Q
    piece = qrows // N_DEV

    def body(x_ref, rw_ref, idx_ref, ew_ref, out_ref,
             xbf_ref, g0_ref, g1_ref, acc_ref,
             rs_send, rs_recv, ag_src, ag_recv,
             rs_send_sems, rs_recv_sems, ag_send_sems, ag_recv_sems):
        q = pl.program_id(0)
        e = pl.program_id(1)
        my = lax.axis_index("i")

        def rs_rdma(qi, k):
            tgt = lax.rem(my + k, N_DEV)
            return pltpu.make_async_remote_copy(
                src_ref=rs_send.at[qi, k - 1],
                dst_ref=rs_recv.at[qi, k - 1],
                send_sem=rs_send_sems.at[qi, k - 1],
                recv_sem=rs_recv_sems.at[qi, k - 1],
                device_id=(tgt,),
                device_id_type=pl.DeviceIdType.MESH,
            )

        def ag_rdma(qi, k):
            tgt = lax.rem(my + k, N_DEV)
            return pltpu.make_async_remote_copy(
                src_ref=ag_src.at[qi],
                dst_ref=ag_recv.at[qi, k - 1],
                send_sem=ag_send_sems.at[qi, k - 1],
                recv_sem=ag_recv_sems.at[qi, k - 1],
                device_id=(tgt,),
                device_id_type=pl.DeviceIdType.MESH,
            )

        def issue_rs(qi):
            for k in range(1, N_DEV):
                tgt = lax.rem(my + k, N_DEV)
                rs_send[qi, k - 1, :, :] = acc_ref[
                    pl.ds(qi * qrows + tgt * piece, piece), :
                ].astype(jnp.bfloat16)
            for k in range(1, N_DEV):
                rs_rdma(qi, k).start()

        def finish_rs_issue_ag(qi):
            for k in range(1, N_DEV):
                rs_rdma(qi, k).wait()
            red = acc_ref[pl.ds(qi * qrows + my * piece, piece), :]
            for k in range(1, N_DEV):
                red = red + rs_recv[qi, k - 1, :, :].astype(jnp.float32)
            out_ref[pl.ds(qi * qrows + my * piece, piece), :] = red
            ag_src[qi, :, :] = red.astype(jnp.bfloat16)
            for k in range(1, N_DEV):
                ag_rdma(qi, k).start()

        def finish_ag(qi):
            for k in range(1, N_DEV):
                ag_rdma(qi, k).wait()
                src_dev = lax.rem(my + (N_DEV - k), N_DEV)
                out_ref[pl.ds(qi * qrows + src_dev * piece, piece), :] = \
                    ag_recv[qi, k - 1, :, :].astype(jnp.float32)

        @pl.when(e == 0)
        def _init():
            xf = x_ref[...]
            xbf_ref[...] = xf.astype(jnp.bfloat16)
            scores = jnp.dot(xf, rw_ref[...],
                             preferred_element_type=jnp.float32)
            m = jnp.max(scores, axis=-1, keepdims=True)
            p = jnp.exp(scores - m)
            iota = lax.broadcasted_iota(jnp.int32, scores.shape, 1)
            p0 = jnp.sum(jnp.where(iota == idx_ref[:, 0:1], p, 0.0),
                         axis=1, keepdims=True)
            p1 = jnp.sum(jnp.where(iota == idx_ref[:, 1:2], p, 0.0),
                         axis=1, keepdims=True)
            s = p0 + p1
            g0_ref[...] = p0 / s
            g1_ref[...] = p1 / s
            acc_ref[pl.ds(q * qrows, qrows), :] = jnp.zeros(
                (qrows, d_out), jnp.float32)

        eid = my * e_loc + e
        w_col = (jnp.where(idx_ref[:, 0:1] == eid, g0_ref[...], 0.0)
                 + jnp.where(idx_ref[:, 1:2] == eid, g1_ref[...], 0.0))
        wblk = ew_ref[0, :, :].astype(jnp.bfloat16)
        y = jnp.dot(xbf_ref[...], wblk, preferred_element_type=jnp.float32)
        acc_ref[pl.ds(q * qrows, qrows), :] += w_col * y

        @pl.when(jnp.logical_and(q == 0, e == e_loc - 1))
        def _barrier():
            barrier_sem = pltpu.get_barrier_semaphore()
            for k in range(1, N_DEV):
                pl.semaphore_signal(
                    barrier_sem, inc=1,
                    device_id=(lax.rem(my + k, N_DEV),),
                    device_id_type=pl.DeviceIdType.MESH)
            pl.semaphore_wait(barrier_sem, N_DEV - 1)

        @pl.when(e == e_loc - 1)
        def _issue_rs_q():
            issue_rs(q)

        @pl.when(jnp.logical_and(q >= 1, e == 3))
        def _mid():
            finish_rs_issue_ag(q - 1)

        @pl.when(jnp.logical_and(q >= 1, e == e_loc - 1))
        def _end():
            finish_ag(q - 1)

        @pl.when(jnp.logical_and(q == N_Q - 1, e == e_loc - 1))
        def _tail():
            finish_rs_issue_ag(N_Q - 1)
            finish_ag(N_Q - 1)

    grid = (N_Q, e_loc)
    return pl.pallas_call(
        body,
        grid=grid,
        out_shape=jax.ShapeDtypeStruct((n_tok, d_out), jnp.float32),
        in_specs=[
            pl.BlockSpec((qrows, d_in), lambda q, e: (q, 0)),
            pl.BlockSpec(router_W.shape, lambda q, e: (0, 0)),
            pl.BlockSpec((qrows, 2), lambda q, e: (q, 0)),
            pl.BlockSpec((1,) + expert_W.shape[1:], lambda q, e: (e, 0, 0)),
        ],
        out_specs=pl.BlockSpec((n_tok, d_out), lambda q, e: (0, 0)),
        scratch_shapes=[
            pltpu.VMEM((qrows, d_in), jnp.bfloat16),
            pltpu.VMEM((qrows, 1), jnp.float32),
            pltpu.VMEM((qrows, 1), jnp.float32),
            pltpu.VMEM((n_tok, d_out), jnp.float32),
            pltpu.VMEM((N_Q, N_DEV - 1, piece, d_out), jnp.bfloat16),
            pltpu.VMEM((N_Q, N_DEV - 1, piece, d_out), jnp.bfloat16),
            pltpu.VMEM((N_Q, piece, d_out), jnp.bfloat16),
            pltpu.VMEM((N_Q, N_DEV - 1, piece, d_out), jnp.bfloat16),
            pltpu.SemaphoreType.DMA((N_Q, N_DEV - 1)),
            pltpu.SemaphoreType.DMA((N_Q, N_DEV - 1)),
            pltpu.SemaphoreType.DMA((N_Q, N_DEV - 1)),
            pltpu.SemaphoreType.DMA((N_Q, N_DEV - 1)),
        ],
        compiler_params=pltpu.CompilerParams(
            collective_id=0,
            dimension_semantics=("arbitrary", "arbitrary"),
            vmem_limit_bytes=100 * 1024 * 1024,
        ),
    )(x, router_W, route_idx, expert_W)
